# Initial kernel scaffold; baseline (speedup 1.0000x reference)
import jax
import jax.numpy as jnp
from jax.experimental import pallas as pl


def kernel(x, r_ij, neighbors, neighbor_mask, f_ij, W1, b1, W2, b2, Wi, Wf, bf, Wd, bd):
    raise NotImplementedError("write your pallas kernel here")


# speedup vs baseline: 8.2611x; 8.2611x over previous
import jax
import jax.numpy as jnp
from jax.experimental import pallas as pl
from jax.experimental.pallas import tpu as pltpu

_B, _N, _NBH = 8, 512, 32
_AB, _SB, _NF = 256, 64, 256
_BN = 64
_NBLK = _N // _BN
_E = _BN * _NBH


def _ssp(v):
    return jnp.logaddexp(v, 0.0) - jnp.log(2.0)


def _block_kernel(x_ref, nbr_ref, mask_ref, f_ref,
                  W1_ref, b1_ref, W2_ref, b2_ref,
                  Wi_ref, Wf_ref, bf_ref, Wd_ref, bd_ref,
                  out_ref, y_scr):
    nb = pl.program_id(1)

    @pl.when(nb == 0)
    def _():
        y_scr[:] = jnp.dot(x_ref[0], Wi_ref[:],
                           preferred_element_type=jnp.float32)

    f = f_ref[0, 0]
    h = _ssp(jnp.dot(f, W1_ref[:], preferred_element_type=jnp.float32)
             + b1_ref[:])
    wfilt = jnp.dot(h, W2_ref[:], preferred_element_type=jnp.float32) + b2_ref[:]
    wfilt = wfilt * mask_ref[0, 0]

    idx = nbr_ref[0, 0]
    onehot = (idx == jax.lax.broadcasted_iota(jnp.int32, (_E, _N), 1)
              ).astype(jnp.float32)
    y_nbh = jnp.dot(onehot, y_scr[:], preferred_element_type=jnp.float32)

    agg = (y_nbh * wfilt).reshape(_BN, _NBH, _NF).sum(axis=1)
    v = _ssp(jnp.dot(agg, Wf_ref[:], preferred_element_type=jnp.float32)
             + bf_ref[:])
    out_ref[0] = jnp.dot(v, Wd_ref[:], preferred_element_type=jnp.float32) + bd_ref[:]


def kernel(x, r_ij, neighbors, neighbor_mask, f_ij,
           W1, b1, W2, b2, Wi, Wf, bf, Wd, bd):
    del r_ij
    grid = (_B, _NBLK)
    full = lambda shape: pl.BlockSpec(shape, lambda b, nb: (0,) * len(shape))

    nbr_r = neighbors.reshape(_B, _NBLK, _E, 1)
    mask_r = neighbor_mask.reshape(_B, _NBLK, _E, 1)
    f_r = f_ij.reshape(_B, _NBLK, _E, _SB)

    out = pl.pallas_call(
        _block_kernel,
        grid=grid,
        in_specs=[
            pl.BlockSpec((1, _N, _AB), lambda b, nb: (b, 0, 0)),
            pl.BlockSpec((1, 1, _E, 1), lambda b, nb: (b, nb, 0, 0)),
            pl.BlockSpec((1, 1, _E, 1), lambda b, nb: (b, nb, 0, 0)),
            pl.BlockSpec((1, 1, _E, _SB), lambda b, nb: (b, nb, 0, 0)),
            full((_SB, _NF)),
            full((1, _NF)),
            full((_NF, _NF)),
            full((1, _NF)),
            full((_AB, _NF)),
            full((_NF, _AB)),
            full((1, _AB)),
            full((_AB, _AB)),
            full((1, _AB)),
        ],
        out_specs=pl.BlockSpec((1, _BN, _AB), lambda b, nb: (b, nb, 0)),
        out_shape=jax.ShapeDtypeStruct((_B, _N, _AB), jnp.float32),
        scratch_shapes=[pltpu.VMEM((_N, _NF), jnp.float32)],
        compiler_params=pltpu.CompilerParams(
            dimension_semantics=("parallel", "arbitrary"),
        ),
    )(x, nbr_r, mask_r, f_r,
      W1, b1.reshape(1, _NF), W2, b2.reshape(1, _NF),
      Wi, Wf, bf.reshape(1, _AB), Wd, bd.reshape(1, _AB))
    return out
